# baseline (speedup 1.0000x reference)
import functools

import jax
import jax.numpy as jnp
from jax.experimental import pallas as pl

N = 4096
BM = 128


def _scaled_rhs_kernel(x_ref, w_ref, wb_ref, out_ref):
    z = jnp.dot(x_ref[...], w_ref[...], preferred_element_type=jnp.float32)
    out_ref[0] = wb_ref[0, 0] * z
    out_ref[1] = wb_ref[1, 0] * z


def _pass1_kernel(a_ref, zs_ref, b1_ref, u1_ref):
    acc = jnp.dot(a_ref[0], zs_ref[0], preferred_element_type=jnp.float32)
    acc += jnp.dot(a_ref[1], zs_ref[1], preferred_element_type=jnp.float32)
    u1_ref[...] = jnp.maximum(acc + b1_ref[...], 0.0)


def _pass2_kernel(a_ref, vs_ref, u1_ref, b2_ref, out_ref):
    acc = jnp.dot(a_ref[0], vs_ref[0], preferred_element_type=jnp.float32)
    acc += jnp.dot(a_ref[1], vs_ref[1], preferred_element_type=jnp.float32)
    out_ref[...] = 0.5 * (u1_ref[...] + acc + b2_ref[...])


@jax.jit
def kernel(feature, A, W1, b1, W2, b2, weight_b):
    n = A.shape[1]
    hid = W1.shape[1]
    out_dim = W2.shape[1]
    grid = (n // BM,)

    a_spec = pl.BlockSpec((2, BM, n), lambda i: (0, i, 0))
    full2 = lambda d: pl.BlockSpec((2, n, d), lambda i: (0, 0, 0))
    row_spec = lambda d: pl.BlockSpec((BM, d), lambda i: (i, 0))
    bias_spec = lambda d: pl.BlockSpec((1, d), lambda i: (0, 0))

    zs = pl.pallas_call(
        _scaled_rhs_kernel,
        out_shape=jax.ShapeDtypeStruct((2, n, hid), jnp.float32),
    )(feature, W1, weight_b)

    u1 = pl.pallas_call(
        _pass1_kernel,
        grid=grid,
        in_specs=[a_spec, full2(hid), bias_spec(hid)],
        out_specs=row_spec(hid),
        out_shape=jax.ShapeDtypeStruct((n, hid), jnp.float32),
    )(A, zs, b1.reshape(1, hid))

    vs = pl.pallas_call(
        _scaled_rhs_kernel,
        out_shape=jax.ShapeDtypeStruct((2, n, out_dim), jnp.float32),
    )(u1, W2, weight_b)

    out = pl.pallas_call(
        _pass2_kernel,
        grid=grid,
        in_specs=[a_spec, full2(out_dim), row_spec(hid), bias_spec(out_dim)],
        out_specs=row_spec(out_dim),
        out_shape=jax.ShapeDtypeStruct((n, out_dim), jnp.float32),
    )(A, vs, u1, b2.reshape(1, out_dim))

    return out


# speedup vs baseline: 1.2352x; 1.2352x over previous
import functools

import jax
import jax.numpy as jnp
import numpy as np
from jax.experimental import pallas as pl
from jax.experimental.pallas import tpu as pltpu

N = 4096
B = 1024
C = N // B


def _scaled_rhs_kernel(x_ref, w_ref, wb_ref, out_ref):
    z = jnp.dot(x_ref[...], w_ref[...], preferred_element_type=jnp.float32)
    out_ref[0] = wb_ref[0, 0] * z
    out_ref[1] = wb_ref[1, 0] * z


def _main_kernel(a_ref, zs_ref, w2_ref, wb_ref, b1_ref,
                 u1_ref, u2p_ref, vs_out_ref,
                 u1_acc, u2_acc, vs_scr):
    r = pl.program_id(0)
    c = pl.program_id(1)

    @pl.when(c == 0)
    def _init():
        u1_acc[...] = jnp.zeros_like(u1_acc)
        u2_acc[...] = jnp.zeros_like(u2_acc)

    u1_acc[...] += (
        jnp.dot(a_ref[0], zs_ref[0], preferred_element_type=jnp.float32)
        + jnp.dot(a_ref[1], zs_ref[1], preferred_element_type=jnp.float32)
    )

    @pl.when(c < r)
    def _salvage():
        v0 = vs_scr[0, pl.ds(c * B, B), :]
        v1 = vs_scr[1, pl.ds(c * B, B), :]
        u2_acc[...] += (
            jnp.dot(a_ref[0], v0, preferred_element_type=jnp.float32)
            + jnp.dot(a_ref[1], v1, preferred_element_type=jnp.float32)
        )

    @pl.when(c == C - 1)
    def _finalize_row():
        u1 = jnp.maximum(u1_acc[...] + b1_ref[...], 0.0)
        u1_ref[...] = u1
        v = jnp.dot(u1, w2_ref[...], preferred_element_type=jnp.float32)
        v0 = wb_ref[0, 0] * v
        v1 = wb_ref[1, 0] * v
        vs_scr[0, pl.ds(r * B, B), :] = v0
        vs_scr[1, pl.ds(r * B, B), :] = v1
        vs_out_ref[0] = v0
        vs_out_ref[1] = v1
        u2p_ref[...] = u2_acc[...]


def _remainder_kernel(rmap_ref, cmap_ref, a_ref, vs_ref, u1_ref, u2p_ref,
                      b2_ref, out_ref, acc_ref):
    t = pl.program_id(0)
    r = rmap_ref[t]
    c = cmap_ref[t]

    @pl.when(c == r)
    def _init():
        acc_ref[...] = u2p_ref[...]

    acc_ref[...] += (
        jnp.dot(a_ref[0], vs_ref[0], preferred_element_type=jnp.float32)
        + jnp.dot(a_ref[1], vs_ref[1], preferred_element_type=jnp.float32)
    )

    @pl.when(c == C - 1)
    def _final():
        out_ref[...] = 0.5 * (u1_ref[...] + acc_ref[...] + b2_ref[...])


@jax.jit
def kernel(feature, A, W1, b1, W2, b2, weight_b):
    n = A.shape[1]
    hid = W1.shape[1]
    out_dim = W2.shape[1]

    zs = pl.pallas_call(
        _scaled_rhs_kernel,
        out_shape=jax.ShapeDtypeStruct((2, n, hid), jnp.float32),
    )(feature, W1, weight_b)

    a_spec = pl.BlockSpec((2, B, B), lambda r, c: (0, r, c))
    u1, u2p, vs = pl.pallas_call(
        _main_kernel,
        grid=(C, C),
        in_specs=[
            a_spec,
            pl.BlockSpec((2, B, hid), lambda r, c: (0, c, 0)),
            pl.BlockSpec((hid, out_dim), lambda r, c: (0, 0)),
            pl.BlockSpec((2, 1), lambda r, c: (0, 0)),
            pl.BlockSpec((1, hid), lambda r, c: (0, 0)),
        ],
        out_specs=[
            pl.BlockSpec((B, hid), lambda r, c: (r, 0)),
            pl.BlockSpec((B, out_dim), lambda r, c: (r, 0)),
            pl.BlockSpec((2, B, out_dim), lambda r, c: (0, r, 0)),
        ],
        out_shape=[
            jax.ShapeDtypeStruct((n, hid), jnp.float32),
            jax.ShapeDtypeStruct((n, out_dim), jnp.float32),
            jax.ShapeDtypeStruct((2, n, out_dim), jnp.float32),
        ],
        scratch_shapes=[
            pltpu.VMEM((B, hid), jnp.float32),
            pltpu.VMEM((B, out_dim), jnp.float32),
            pltpu.VMEM((2, n, out_dim), jnp.float32),
        ],
    )(A, zs, W2, weight_b, b1.reshape(1, hid))

    pairs = [(r, c) for r in range(C) for c in range(r, C)]
    rmap = jnp.asarray(np.array([p[0] for p in pairs], dtype=np.int32))
    cmap = jnp.asarray(np.array([p[1] for p in pairs], dtype=np.int32))

    grid_spec = pltpu.PrefetchScalarGridSpec(
        num_scalar_prefetch=2,
        grid=(len(pairs),),
        in_specs=[
            pl.BlockSpec((2, B, B), lambda t, rm, cm: (0, rm[t], cm[t])),
            pl.BlockSpec((2, B, out_dim), lambda t, rm, cm: (0, cm[t], 0)),
            pl.BlockSpec((B, hid), lambda t, rm, cm: (rm[t], 0)),
            pl.BlockSpec((B, out_dim), lambda t, rm, cm: (rm[t], 0)),
            pl.BlockSpec((1, out_dim), lambda t, rm, cm: (0, 0)),
        ],
        out_specs=pl.BlockSpec((B, out_dim), lambda t, rm, cm: (rm[t], 0)),
        scratch_shapes=[pltpu.VMEM((B, out_dim), jnp.float32)],
    )

    out = pl.pallas_call(
        _remainder_kernel,
        grid_spec=grid_spec,
        out_shape=jax.ShapeDtypeStruct((n, out_dim), jnp.float32),
    )(rmap, cmap, A, vs, u1, u2p, b2.reshape(1, out_dim))

    return out


# speedup vs baseline: 1.2462x; 1.0089x over previous
import functools

import jax
import jax.numpy as jnp
import numpy as np
from jax.experimental import pallas as pl
from jax.experimental.pallas import tpu as pltpu

N = 4096
BM = 256
CW = 1024
NC = N // CW
RG = CW // BM


def _scaled_rhs_kernel(x_ref, w_ref, wb_ref, out_ref):
    z = jnp.dot(x_ref[...], w_ref[...], preferred_element_type=jnp.float32)
    out_ref[0] = wb_ref[0, 0] * z
    out_ref[1] = wb_ref[1, 0] * z


def _main_kernel(a_ref, zs_ref, w2_ref, wb_ref, b1_ref,
                 u1_ref, u2p_ref, vs_out_ref, vs_scr):
    r = pl.program_id(0)

    acc = (
        jnp.dot(a_ref[0], zs_ref[0], preferred_element_type=jnp.float32)
        + jnp.dot(a_ref[1], zs_ref[1], preferred_element_type=jnp.float32)
    )
    u1 = jnp.maximum(acc + b1_ref[...], 0.0)
    u1_ref[...] = u1
    v = jnp.dot(u1, w2_ref[...], preferred_element_type=jnp.float32)
    v0 = wb_ref[0, 0] * v
    v1 = wb_ref[1, 0] * v
    vs_scr[0, pl.ds(r * BM, BM), :] = v0
    vs_scr[1, pl.ds(r * BM, BM), :] = v1
    vs_out_ref[0] = v0
    vs_out_ref[1] = v1

    u2p_ref[...] = jnp.zeros_like(u2p_ref)
    for c in range(NC - 1):
        @pl.when(c < r // RG)
        def _salvage(c=c):
            sl = slice(c * CW, (c + 1) * CW)
            u2p_ref[...] += (
                jnp.dot(a_ref[0][:, sl], vs_scr[0, sl, :],
                        preferred_element_type=jnp.float32)
                + jnp.dot(a_ref[1][:, sl], vs_scr[1, sl, :],
                          preferred_element_type=jnp.float32)
            )


def _remainder_kernel(rmap_ref, cmap_ref, a_ref, vs_ref, u1_ref, u2p_ref,
                      b2_ref, out_ref, acc_ref):
    t = pl.program_id(0)
    r = rmap_ref[t]
    c = cmap_ref[t]

    @pl.when(c == r)
    def _init():
        acc_ref[...] = u2p_ref[...]

    acc_ref[...] += (
        jnp.dot(a_ref[0], vs_ref[0], preferred_element_type=jnp.float32)
        + jnp.dot(a_ref[1], vs_ref[1], preferred_element_type=jnp.float32)
    )

    @pl.when(c == NC - 1)
    def _final():
        out_ref[...] = 0.5 * (u1_ref[...] + acc_ref[...] + b2_ref[...])


@jax.jit
def kernel(feature, A, W1, b1, W2, b2, weight_b):
    n = A.shape[1]
    hid = W1.shape[1]
    out_dim = W2.shape[1]

    zs = pl.pallas_call(
        _scaled_rhs_kernel,
        out_shape=jax.ShapeDtypeStruct((2, n, hid), jnp.float32),
    )(feature, W1, weight_b)

    u1, u2p, vs = pl.pallas_call(
        _main_kernel,
        grid=(n // BM,),
        in_specs=[
            pl.BlockSpec((2, BM, n), lambda r: (0, r, 0)),
            pl.BlockSpec((2, n, hid), lambda r: (0, 0, 0)),
            pl.BlockSpec((hid, out_dim), lambda r: (0, 0)),
            pl.BlockSpec((2, 1), lambda r: (0, 0)),
            pl.BlockSpec((1, hid), lambda r: (0, 0)),
        ],
        out_specs=[
            pl.BlockSpec((BM, hid), lambda r: (r, 0)),
            pl.BlockSpec((BM, out_dim), lambda r: (r, 0)),
            pl.BlockSpec((2, BM, out_dim), lambda r: (0, r, 0)),
        ],
        out_shape=[
            jax.ShapeDtypeStruct((n, hid), jnp.float32),
            jax.ShapeDtypeStruct((n, out_dim), jnp.float32),
            jax.ShapeDtypeStruct((2, n, out_dim), jnp.float32),
        ],
        scratch_shapes=[
            pltpu.VMEM((2, n, out_dim), jnp.float32),
        ],
    )(A, zs, W2, weight_b, b1.reshape(1, hid))

    nc = n // CW
    pairs = [(r, c) for r in range(nc) for c in range(r, nc)]
    rmap = jnp.asarray(np.array([p[0] for p in pairs], dtype=np.int32))
    cmap = jnp.asarray(np.array([p[1] for p in pairs], dtype=np.int32))

    grid_spec = pltpu.PrefetchScalarGridSpec(
        num_scalar_prefetch=2,
        grid=(len(pairs),),
        in_specs=[
            pl.BlockSpec((2, CW, CW), lambda t, rm, cm: (0, rm[t], cm[t])),
            pl.BlockSpec((2, CW, out_dim), lambda t, rm, cm: (0, cm[t], 0)),
            pl.BlockSpec((CW, hid), lambda t, rm, cm: (rm[t], 0)),
            pl.BlockSpec((CW, out_dim), lambda t, rm, cm: (rm[t], 0)),
            pl.BlockSpec((1, out_dim), lambda t, rm, cm: (0, 0)),
        ],
        out_specs=pl.BlockSpec((CW, out_dim), lambda t, rm, cm: (rm[t], 0)),
        scratch_shapes=[pltpu.VMEM((CW, out_dim), jnp.float32)],
    )

    out = pl.pallas_call(
        _remainder_kernel,
        grid_spec=grid_spec,
        out_shape=jax.ShapeDtypeStruct((n, out_dim), jnp.float32),
    )(rmap, cmap, A, vs, u1, u2p, b2.reshape(1, out_dim))

    return out
